# baseline (speedup 1.0000x reference)
import functools

import jax
import jax.numpy as jnp
from jax import lax
from jax.experimental import pallas as pl
from jax.experimental.pallas import tpu as pltpu
from jax.experimental.pallas import tpu_sc as plsc

N = 320000
NUM_SEG = 10000
Q_SIZE = 128
K_SIZE = 128
HID = 64
D_V = 128

NSLICE = 2
NS = N // NSLICE


BLK = 6400
GRID1 = NS // BLK
EROWS = BLK // 128


def _tc_dense_body(q_ref, k_ref, v_ref, w_ref, u_ref, vw_ref, p_ref, e_ref):
    qw = jnp.dot(q_ref[...], w_ref[...], preferred_element_type=jnp.float32)
    ku = jnp.dot(k_ref[...], u_ref[...], preferred_element_type=jnp.float32)
    t = jnp.tanh(qw + ku)
    beta = jnp.sum(t * vw_ref[0:1, :], axis=1, keepdims=True)
    e = jnp.exp(beta)
    p_ref[...] = v_ref[...] * e
    e_ref[...] = jnp.reshape(e[:, 0], (1, EROWS, 128))


def _tc_dense(Q, K, V, W_w, U_w, vw8, s):
    off = s * GRID1
    return pl.pallas_call(
        _tc_dense_body,
        grid=(GRID1,),
        in_specs=[
            pl.BlockSpec((BLK, Q_SIZE), lambda i: (i + off, 0)),
            pl.BlockSpec((BLK, K_SIZE), lambda i: (i + off, 0)),
            pl.BlockSpec((BLK, D_V), lambda i: (i + off, 0)),
            pl.BlockSpec((Q_SIZE, HID), lambda i: (0, 0)),
            pl.BlockSpec((K_SIZE, HID), lambda i: (0, 0)),
            pl.BlockSpec((8, HID), lambda i: (0, 0)),
        ],
        out_specs=[
            pl.BlockSpec((BLK, D_V), lambda i: (i, 0)),
            pl.BlockSpec((1, EROWS, 128), lambda i: (i, 0, 0)),
        ],
        out_shape=[
            jax.ShapeDtypeStruct((NS, D_V), jnp.float32),
            jax.ShapeDtypeStruct((GRID1, EROWS, 128), jnp.float32),
        ],
    )(Q, K, V, W_w, U_w, vw8)


NCORE = 2
NSUB = 16
NW = NCORE * NSUB
CH = 80
NCHUNK = NS // CH
TOT = -(-NCHUNK // NW)
NBUF = 3
PREF = 2
LAG = NBUF - PREF
NROUND = -(-TOT // NBUF)
DEN_ROWS = 80
SEG_PER_TILE = 624
SEG_TAIL = NUM_SEG - NSUB * SEG_PER_TILE
ZR = 104


def _sc_scatter_body(p_hbm, e_hbm, idx_hbm, zeros_hbm,
                     acc_out, den_out, *rest):
    data = list(rest[0:NBUF])
    ev = list(rest[NBUF:2 * NBUF])
    xv = list(rest[2 * NBUF:3 * NBUF])
    den_v = rest[3 * NBUF]
    idx_id = rest[3 * NBUF + 1]
    acc_sh = rest[3 * NBUF + 2]
    den_sh = rest[3 * NBUF + 3]
    isem = list(rest[3 * NBUF + 4:4 * NBUF + 4])
    ssem = list(rest[4 * NBUF + 4:5 * NBUF + 4])

    cid = lax.axis_index("c")
    sid = lax.axis_index("s")
    wid = cid * NSUB + sid

    def valid(i):
        return (i * NW + wid) < NCHUNK

    def in_copies(i, b):
        base = (i * NW + wid) * CH
        return (
            pltpu.make_async_copy(p_hbm.at[pl.ds(base, CH)], data[b], isem[b]),
            pltpu.make_async_copy(e_hbm.at[pl.ds(base, CH)], ev[b], isem[b]),
            pltpu.make_async_copy(idx_hbm.at[pl.ds(base, CH)], xv[b], isem[b]),
        )

    def scat_copy(b):
        return pltpu.make_async_copy(data[b], acc_sh.at[xv[b]], ssem[b])

    for z in range(SEG_PER_TILE // ZR):
        pltpu.sync_copy(zeros_hbm,
                        acc_sh.at[pl.ds(sid * SEG_PER_TILE + z * ZR, ZR)])

    @pl.when(sid == NSUB - 1)
    def _():
        pltpu.sync_copy(zeros_hbm.at[pl.ds(0, SEG_TAIL)],
                        acc_sh.at[pl.ds(NSUB * SEG_PER_TILE, SEG_TAIL)])

    pltpu.sync_copy(zeros_hbm.at[pl.ds(0, DEN_ROWS)], den_v)
    for j in range(DEN_ROWS // 16):
        idx_id[pl.ds(j * 16, 16)] = lax.iota(jnp.int32, 16) + (j * 16)

    @pl.when(sid == 0)
    def _():
        pltpu.sync_copy(zeros_hbm.at[pl.ds(0, DEN_ROWS)], den_sh)

    plsc.subcore_barrier()

    for b in range(PREF):
        for c in in_copies(b, b):
            c.start()

    def _round(o, carry):
        for b in range(NBUF):
            i = o * NBUF + b

            @pl.when(valid(i))
            def _():
                for c in in_copies(i, b):
                    c.wait()
                for j in range(CH // 16):
                    sl = pl.ds(j * 16, 16)
                    idx16 = xv[b][sl]
                    hi16 = lax.shift_right_logical(idx16, 7)
                    lo16 = lax.bitwise_and(idx16, 127)
                    plsc.addupdate_scatter(den_v, [hi16, lo16], ev[b][sl])
                scat_copy(b).start(add=True)

            bp = (b + PREF) % NBUF

            @pl.when((i >= LAG) & valid(i - LAG))
            def _():
                scat_copy(bp).wait()

            @pl.when(valid(i + PREF))
            def _():
                for c in in_copies(i + PREF, bp):
                    c.start()
        return carry

    lax.fori_loop(0, NROUND, _round, 0)

    for i in range(NROUND * NBUF - LAG, NROUND * NBUF):
        @pl.when(valid(i))
        def _():
            scat_copy(i % NBUF).wait()

    pltpu.sync_copy(den_v, den_sh.at[idx_id], add=True)

    plsc.subcore_barrier()

    row0 = sid * SEG_PER_TILE
    pltpu.sync_copy(acc_sh.at[pl.ds(row0, SEG_PER_TILE)],
                    acc_out.at[cid, pl.ds(row0, SEG_PER_TILE)])

    @pl.when(sid == NSUB - 1)
    def _():
        pltpu.sync_copy(acc_sh.at[pl.ds(NSUB * SEG_PER_TILE, SEG_TAIL)],
                        acc_out.at[cid, pl.ds(NSUB * SEG_PER_TILE, SEG_TAIL)])

    @pl.when(sid == 0)
    def _():
        pltpu.sync_copy(den_sh, den_out.at[cid])


def _sc_scatter(P, E1, idx1, zeros):
    f = functools.partial(
        pl.kernel,
        mesh=plsc.VectorSubcoreMesh(core_axis_name="c", subcore_axis_name="s"),
        compiler_params=pltpu.CompilerParams(needs_layout_passes=False),
        out_type=[
            jax.ShapeDtypeStruct((NCORE, NUM_SEG, D_V), jnp.float32),
            jax.ShapeDtypeStruct((NCORE, DEN_ROWS, 128), jnp.float32),
        ],
        scratch_types=(
            [pltpu.VMEM((CH, D_V), jnp.float32) for _ in range(NBUF)]
            + [pltpu.VMEM((CH,), jnp.float32) for _ in range(NBUF)]
            + [pltpu.VMEM((CH,), jnp.int32) for _ in range(NBUF)]
            + [pltpu.VMEM((DEN_ROWS, 128), jnp.float32),
               pltpu.VMEM((DEN_ROWS,), jnp.int32),
               pltpu.VMEM_SHARED((NUM_SEG, D_V), jnp.float32),
               pltpu.VMEM_SHARED((DEN_ROWS, 128), jnp.float32)]
            + [pltpu.SemaphoreType.DMA for _ in range(2 * NBUF)]
        ),
    )(_sc_scatter_body)
    return f(P, E1, idx1, zeros)


def _tc_combine_body(acc0_ref, acc1_ref, den0_ref, den1_ref, out_ref):
    num = (acc0_ref[0] + acc0_ref[1]) + (acc1_ref[0] + acc1_ref[1])
    den = (jnp.sum(den0_ref[...], axis=0)
           + jnp.sum(den1_ref[...], axis=0))[:, None]
    out_ref[...] = jnp.where(den > 0.0, num / den, 0.0)


def _tc_combine(acc0, acc1, den0, den1):
    return pl.pallas_call(
        _tc_combine_body,
        out_shape=jax.ShapeDtypeStruct((NUM_SEG, D_V), jnp.float32),
    )(acc0, acc1, den0, den1)


def kernel(Q, K, V, W_w, U_w, V_w, batch_index):
    vw8 = jnp.broadcast_to(V_w.reshape(1, HID), (8, HID))
    idx = batch_index.astype(jnp.int32).reshape(NSLICE, NS)
    zeros = jnp.zeros((ZR, D_V), jnp.float32)
    accs, dens = [], []
    for s in range(NSLICE):
        P, E2 = _tc_dense(Q, K, V, W_w, U_w, vw8, s)
        acc, den = _sc_scatter(P, E2.reshape(NS), idx[s], zeros)
        accs.append(acc)
        dens.append(den.reshape(NCORE, DEN_ROWS * 128)[:, :NUM_SEG])
    return _tc_combine(accs[0], accs[1], dens[0], dens[1])
